# Initial kernel scaffold; baseline (speedup 1.0000x reference)
import jax
import jax.numpy as jnp
from jax.experimental import pallas as pl


def kernel(x, edge_index, W1, b1, W2, b2, W3, b3):
    raise NotImplementedError("write your pallas kernel here")


# speedup vs baseline: 4.6695x; 4.6695x over previous
import functools

import jax
import jax.numpy as jnp
from jax import lax
from jax.experimental import pallas as pl
from jax.experimental.pallas import tpu as pltpu
from jax.experimental.pallas import tpu_sc as plsc

_NC = 2
_NS = 16
_CHUNK = 80


@functools.lru_cache(maxsize=None)
def _sc_segment_sum(n, e, dc):
    nw = _NC * _NS
    e_per_w = e // nw
    n_chunks = e_per_w // _CHUNK
    zrows = 128
    n_pad = -(-n // (_NS * zrows)) * (_NS * zrows)
    rows_per_tile = n_pad // _NS
    n_zcopies = rows_per_tile // zrows
    mesh = plsc.VectorSubcoreMesh(core_axis_name="c", subcore_axis_name="s")

    @functools.partial(
        pl.kernel,
        mesh=mesh,
        out_type=jax.ShapeDtypeStruct((_NC, n_pad, dc), jnp.float32),
        scratch_types=[
            pltpu.VMEM((_CHUNK,), jnp.int32),
            pltpu.VMEM((_CHUNK,), jnp.int32),
            pltpu.VMEM((_CHUNK, dc), jnp.float32),
            pltpu.VMEM((zrows, dc), jnp.float32),
            pltpu.VMEM_SHARED((n, dc), jnp.float32),
        ],
    )
    def k(table_hbm, src_hbm, dst_hbm, out_hbm, src_v, dst_v, rows_v, zero_v, agg_sh):
        cid = lax.axis_index("c")
        sid = lax.axis_index("s")
        wid = sid * _NC + cid

        zvec = jnp.zeros((16,), jnp.float32)
        vecs_per_row = dc // 16

        def zbody(i, carry):
            zero_v[i // vecs_per_row, pl.ds((i % vecs_per_row) * 16, 16)] = zvec
            return carry

        lax.fori_loop(0, zrows * vecs_per_row, zbody, 0)
        row0 = sid * rows_per_tile
        for z in range(n_zcopies):
            pltpu.sync_copy(zero_v, agg_sh.at[pl.ds(row0 + z * zrows, zrows)])
        plsc.subcore_barrier()

        base = wid * e_per_w

        def body(j, carry):
            off = base + j * _CHUNK
            pltpu.sync_copy(src_hbm.at[pl.ds(off, _CHUNK)], src_v)
            pltpu.sync_copy(dst_hbm.at[pl.ds(off, _CHUNK)], dst_v)
            pltpu.sync_copy(table_hbm.at[src_v], rows_v)
            pltpu.sync_copy(rows_v, agg_sh.at[dst_v], add=True)
            return carry

        lax.fori_loop(0, n_chunks, body, 0)
        plsc.subcore_barrier()
        pltpu.sync_copy(agg_sh.at[pl.ds(row0, rows_per_tile)],
                        out_hbm.at[cid, pl.ds(row0, rows_per_tile)])

    return k


@functools.lru_cache(maxsize=None)
def _sc_degree(n, e, dc):
    nw = _NC * _NS
    e_per_w = e // nw
    n_chunks = e_per_w // _CHUNK
    zrows = 128
    n_pad = -(-n // (_NS * zrows)) * (_NS * zrows)
    rows_per_tile = n_pad // _NS
    n_zcopies = rows_per_tile // zrows
    mesh = plsc.VectorSubcoreMesh(core_axis_name="c", subcore_axis_name="s")

    @functools.partial(
        pl.kernel,
        mesh=mesh,
        out_type=jax.ShapeDtypeStruct((_NC, n_pad, dc), jnp.float32),
        scratch_types=[
            pltpu.VMEM((_CHUNK,), jnp.int32),
            pltpu.VMEM((_CHUNK, dc), jnp.float32),
            pltpu.VMEM((zrows, dc), jnp.float32),
            pltpu.VMEM_SHARED((n_pad, dc), jnp.float32),
        ],
    )
    def k(dst_hbm, out_hbm, dst_v, ones_v, zero_v, deg_sh):
        cid = lax.axis_index("c")
        sid = lax.axis_index("s")
        wid = sid * _NC + cid
        zvec = jnp.zeros((16,), jnp.float32)
        ovec = jnp.ones((16,), jnp.float32)
        vecs_per_row = dc // 16

        def fbody(i, carry):
            zero_v[i // vecs_per_row, pl.ds((i % vecs_per_row) * 16, 16)] = zvec
            return carry

        lax.fori_loop(0, zrows * vecs_per_row, fbody, 0)

        def obody(i, carry):
            ones_v[i // vecs_per_row, pl.ds((i % vecs_per_row) * 16, 16)] = ovec
            return carry

        lax.fori_loop(0, _CHUNK * vecs_per_row, obody, 0)
        row0 = sid * rows_per_tile
        for z in range(n_zcopies):
            pltpu.sync_copy(zero_v, deg_sh.at[pl.ds(row0 + z * zrows, zrows)])
        plsc.subcore_barrier()

        base = wid * e_per_w

        def body(j, carry):
            pltpu.sync_copy(dst_hbm.at[pl.ds(base + j * _CHUNK, _CHUNK)], dst_v)
            pltpu.sync_copy(ones_v, deg_sh.at[dst_v], add=True)
            return carry

        lax.fori_loop(0, n_chunks, body, 0)
        plsc.subcore_barrier()
        pltpu.sync_copy(deg_sh.at[pl.ds(row0, rows_per_tile)],
                        out_hbm.at[cid, pl.ds(row0, rows_per_tile)])

    return k


def _logsoftmax(v):
    m = jnp.max(v, axis=-1, keepdims=True)
    s = v - m
    return s - jnp.log(jnp.sum(jnp.exp(s), axis=-1, keepdims=True))


def _tc_layer1_body(x_ref, a_ref, dg_ref, w_ref, b_ref, h_ref, dinv_ref, *, d, bn):
    a = a_ref[0] + a_ref[1]
    deg = jnp.max(dg_ref[0] + dg_ref[1], axis=1, keepdims=True)
    dinv = 1.0 / jnp.maximum(deg, 1.0)
    aggn = a * dinv
    out = (jnp.dot(x_ref[...], w_ref[:d, :], precision=lax.Precision.HIGHEST,
                   preferred_element_type=jnp.float32)
           + jnp.dot(aggn, w_ref[d:, :], precision=lax.Precision.HIGHEST,
                     preferred_element_type=jnp.float32)
           + b_ref[...])
    h_ref[...] = jnp.maximum(out, 0.0)
    dinv_ref[...] = jnp.broadcast_to(dinv, (bn, d))


def _tc_layer_body(h_ref, a_ref, dinv_ref, w_ref, b_ref, o_ref, *, d, last):
    aggn = (a_ref[0] + a_ref[1]) * dinv_ref[...]
    out = (jnp.dot(h_ref[...], w_ref[:d, :], precision=lax.Precision.HIGHEST,
                   preferred_element_type=jnp.float32)
           + jnp.dot(aggn, w_ref[d:, :], precision=lax.Precision.HIGHEST,
                     preferred_element_type=jnp.float32)
           + b_ref[...])
    o_ref[...] = _logsoftmax(out) if last else jnp.maximum(out, 0.0)


def _tc_layer1(x, agg, degarr, w, b, *, bn=512):
    n, d = x.shape
    grid = (pl.cdiv(n, bn),)
    return pl.pallas_call(
        functools.partial(_tc_layer1_body, d=d, bn=bn),
        grid=grid,
        in_specs=[
            pl.BlockSpec((bn, d), lambda i: (i, 0)),
            pl.BlockSpec((_NC, bn, d), lambda i: (0, i, 0)),
            pl.BlockSpec((_NC, bn, d), lambda i: (0, i, 0)),
            pl.BlockSpec((2 * d, d), lambda i: (0, 0)),
            pl.BlockSpec((1, d), lambda i: (0, 0)),
        ],
        out_specs=[
            pl.BlockSpec((bn, d), lambda i: (i, 0)),
            pl.BlockSpec((bn, d), lambda i: (i, 0)),
        ],
        out_shape=[
            jax.ShapeDtypeStruct((n, d), jnp.float32),
            jax.ShapeDtypeStruct((n, d), jnp.float32),
        ],
    )(x, agg, degarr, w, b.reshape(1, d))


def _tc_layer(h, agg, dinv, w, b, *, last, bn=512):
    n, d = h.shape
    grid = (pl.cdiv(n, bn),)
    return pl.pallas_call(
        functools.partial(_tc_layer_body, d=d, last=last),
        grid=grid,
        in_specs=[
            pl.BlockSpec((bn, d), lambda i: (i, 0)),
            pl.BlockSpec((_NC, bn, d), lambda i: (0, i, 0)),
            pl.BlockSpec((bn, d), lambda i: (i, 0)),
            pl.BlockSpec((2 * d, d), lambda i: (0, 0)),
            pl.BlockSpec((1, d), lambda i: (0, 0)),
        ],
        out_specs=pl.BlockSpec((bn, d), lambda i: (i, 0)),
        out_shape=jax.ShapeDtypeStruct((n, d), jnp.float32),
    )(h, agg, dinv, w, b.reshape(1, d))


def kernel(x, edge_index, W1, b1, W2, b2, W3, b3):
    n, d = x.shape
    e = edge_index.shape[1]
    src = edge_index[0]
    dst = edge_index[1]

    degarr = _sc_degree(n, e, d)(dst)
    agg1 = _sc_segment_sum(n, e, d)(x, src, dst)
    h1, dinv = _tc_layer1(x, agg1, degarr, W1, b1)
    agg2 = _sc_segment_sum(n, e, d)(h1, src, dst)
    h2 = _tc_layer(h1, agg2, dinv, W2, b2, last=False)
    agg3 = _sc_segment_sum(n, e, d)(h2, src, dst)
    return _tc_layer(h2, agg3, dinv, W3, b3, last=True)


# speedup vs baseline: 10.7014x; 2.2918x over previous
import functools

import jax
import jax.numpy as jnp
from jax import lax
from jax.experimental import pallas as pl
from jax.experimental.pallas import tpu as pltpu
from jax.experimental.pallas import tpu_sc as plsc

_NC = 2
_NS = 16
_CHUNK = 100


def _fill_zero(ref, rows, dc, val):
    vec = jnp.full((16,), val, jnp.float32)
    vecs_per_row = dc // 16

    def fbody(i, carry):
        ref[i // vecs_per_row, pl.ds((i % vecs_per_row) * 16, 16)] = vec
        return carry

    lax.fori_loop(0, rows * vecs_per_row, fbody, 0)


@functools.lru_cache(maxsize=None)
def _sc_segment_sum(n, e, dc):
    nw = _NC * _NS
    e_per_w = e // nw
    n_chunks = e_per_w // _CHUNK
    assert n_chunks % 4 == 0
    zrows = 32
    n_pad = -(-n // (_NS * 128)) * (_NS * 128)
    rows_per_tile = n_pad // _NS
    n_zcopies = rows_per_tile // zrows
    mesh = plsc.VectorSubcoreMesh(core_axis_name="c", subcore_axis_name="s")

    @functools.partial(
        pl.kernel,
        mesh=mesh,
        out_type=jax.ShapeDtypeStruct((_NC, n_pad, dc), jnp.float32),
        scratch_types=[
            pltpu.VMEM((4, _CHUNK), jnp.int32),
            pltpu.VMEM((4, _CHUNK), jnp.int32),
            pltpu.VMEM((_CHUNK, dc), jnp.float32),
            pltpu.VMEM((_CHUNK, dc), jnp.float32),
            pltpu.VMEM((zrows, dc), jnp.float32),
            pltpu.VMEM_SHARED((n_pad, dc), jnp.float32),
            pltpu.SemaphoreType.DMA,
            pltpu.SemaphoreType.DMA,
            pltpu.SemaphoreType.DMA((4,)),
            pltpu.SemaphoreType.DMA((4,)),
        ],
    )
    def k(table_hbm, src_hbm, dst_hbm, out_hbm,
          srcs, dsts, rows_a, rows_b, zero_v, agg_sh,
          gsem_a, gsem_b, isem, dsem):
        cid = lax.axis_index("c")
        sid = lax.axis_index("s")
        wid = sid * _NC + cid

        _fill_zero(zero_v, zrows, dc, 0.0)
        row0 = sid * rows_per_tile
        for z in range(n_zcopies):
            pltpu.sync_copy(zero_v, agg_sh.at[pl.ds(row0 + z * zrows, zrows)])
        plsc.subcore_barrier()

        rows = (rows_a, rows_b)
        gsem = (gsem_a, gsem_b)

        pltpu.sync_copy(src_hbm.at[wid, 0], srcs.at[0])
        pltpu.sync_copy(src_hbm.at[wid, 1], srcs.at[1])
        for s in (2, 3):
            pltpu.async_copy(src_hbm.at[wid, s], srcs.at[s], isem.at[s])
        for s in range(4):
            pltpu.async_copy(dst_hbm.at[wid, s], dsts.at[s], dsem.at[s])
        pltpu.async_copy(table_hbm.at[srcs.at[0]], rows_a, gsem_a)
        pltpu.async_copy(table_hbm.at[srcs.at[1]], rows_b, gsem_b)

        def body(j4, carry):
            for u in range(4):
                j = j4 * 4 + u
                b = u % 2
                pltpu.make_async_copy(table_hbm.at[srcs.at[u]], rows[b],
                                      gsem[b]).wait()
                pltpu.make_async_copy(dst_hbm.at[wid, 0], dsts.at[u],
                                      dsem.at[u]).wait()
                pltpu.sync_copy(rows[b], agg_sh.at[dsts.at[u]], add=True)

                @pl.when(j + 2 < n_chunks)
                def _():
                    s2 = (u + 2) % 4
                    pltpu.make_async_copy(src_hbm.at[wid, 0], srcs.at[s2],
                                          isem.at[s2]).wait()
                    pltpu.async_copy(table_hbm.at[srcs.at[s2]], rows[b], gsem[b])

                @pl.when(j + 4 < n_chunks)
                def _():
                    pltpu.async_copy(src_hbm.at[wid, j + 4], srcs.at[u],
                                     isem.at[u])
                    pltpu.async_copy(dst_hbm.at[wid, j + 4], dsts.at[u],
                                     dsem.at[u])

            return carry

        lax.fori_loop(0, n_chunks // 4, body, 0)
        plsc.subcore_barrier()
        pltpu.sync_copy(agg_sh.at[pl.ds(row0, rows_per_tile)],
                        out_hbm.at[cid, pl.ds(row0, rows_per_tile)])

    return k


@functools.lru_cache(maxsize=None)
def _sc_degree(n, e, dc):
    nw = _NC * _NS
    e_per_w = e // nw
    n_chunks = e_per_w // _CHUNK
    zrows = 32
    n_pad = -(-n // (_NS * 128)) * (_NS * 128)
    rows_per_tile = n_pad // _NS
    n_zcopies = rows_per_tile // zrows
    mesh = plsc.VectorSubcoreMesh(core_axis_name="c", subcore_axis_name="s")

    group = 10
    assert n_chunks % group == 0

    @functools.partial(
        pl.kernel,
        mesh=mesh,
        out_type=jax.ShapeDtypeStruct((_NC, n_pad, dc), jnp.float32),
        scratch_types=[
            pltpu.VMEM((n_chunks, _CHUNK), jnp.int32),
            pltpu.VMEM((_CHUNK, dc), jnp.float32),
            pltpu.VMEM((zrows, dc), jnp.float32),
            pltpu.VMEM_SHARED((n_pad, dc), jnp.float32),
            pltpu.SemaphoreType.DMA,
        ],
    )
    def k(dst_hbm, out_hbm, dsts, ones_v, zero_v, deg_sh, sem):
        cid = lax.axis_index("c")
        sid = lax.axis_index("s")
        wid = sid * _NC + cid
        pltpu.sync_copy(dst_hbm.at[wid], dsts)
        _fill_zero(zero_v, zrows, dc, 0.0)
        _fill_zero(ones_v, _CHUNK, dc, 1.0)
        row0 = sid * rows_per_tile
        for z in range(n_zcopies):
            pltpu.sync_copy(zero_v, deg_sh.at[pl.ds(row0 + z * zrows, zrows)])
        plsc.subcore_barrier()

        def body(g, carry):
            j0 = g * group
            for u in range(group):
                pltpu.async_copy(ones_v, deg_sh.at[dsts.at[j0 + u]], sem, add=True)
            for u in range(group):
                pltpu.make_async_copy(ones_v, deg_sh.at[dsts.at[j0 + u]], sem).wait()
            return carry

        lax.fori_loop(0, n_chunks // group, body, 0)
        plsc.subcore_barrier()
        pltpu.sync_copy(deg_sh.at[pl.ds(row0, rows_per_tile)],
                        out_hbm.at[cid, pl.ds(row0, rows_per_tile)])

    return k


def _logsoftmax(v):
    m = jnp.max(v, axis=-1, keepdims=True)
    s = v - m
    return s - jnp.log(jnp.sum(jnp.exp(s), axis=-1, keepdims=True))


def _tc_layer1_body(x_ref, a_ref, dg_ref, w_ref, b_ref, h_ref, dinv_ref, *, d, bn):
    a = a_ref[0] + a_ref[1]
    deg = jnp.max(dg_ref[0] + dg_ref[1], axis=1, keepdims=True)
    dinv = 1.0 / jnp.maximum(deg, 1.0)
    aggn = a * dinv
    out = (jnp.dot(x_ref[...], w_ref[:d, :], precision=lax.Precision.HIGHEST,
                   preferred_element_type=jnp.float32)
           + jnp.dot(aggn, w_ref[d:, :], precision=lax.Precision.HIGHEST,
                     preferred_element_type=jnp.float32)
           + b_ref[...])
    h_ref[...] = jnp.maximum(out, 0.0)
    dinv_ref[...] = jnp.broadcast_to(dinv, (bn, d))


def _tc_layer_body(h_ref, a_ref, dinv_ref, w_ref, b_ref, o_ref, *, d, last):
    aggn = (a_ref[0] + a_ref[1]) * dinv_ref[...]
    out = (jnp.dot(h_ref[...], w_ref[:d, :], precision=lax.Precision.HIGHEST,
                   preferred_element_type=jnp.float32)
           + jnp.dot(aggn, w_ref[d:, :], precision=lax.Precision.HIGHEST,
                     preferred_element_type=jnp.float32)
           + b_ref[...])
    o_ref[...] = _logsoftmax(out) if last else jnp.maximum(out, 0.0)


def _tc_layer1(x, agg, degarr, w, b, *, bn=512):
    n, d = x.shape
    grid = (pl.cdiv(n, bn),)
    return pl.pallas_call(
        functools.partial(_tc_layer1_body, d=d, bn=bn),
        grid=grid,
        in_specs=[
            pl.BlockSpec((bn, d), lambda i: (i, 0)),
            pl.BlockSpec((_NC, bn, d), lambda i: (0, i, 0)),
            pl.BlockSpec((_NC, bn, d), lambda i: (0, i, 0)),
            pl.BlockSpec((2 * d, d), lambda i: (0, 0)),
            pl.BlockSpec((1, d), lambda i: (0, 0)),
        ],
        out_specs=[
            pl.BlockSpec((bn, d), lambda i: (i, 0)),
            pl.BlockSpec((bn, d), lambda i: (i, 0)),
        ],
        out_shape=[
            jax.ShapeDtypeStruct((n, d), jnp.float32),
            jax.ShapeDtypeStruct((n, d), jnp.float32),
        ],
    )(x, agg, degarr, w, b.reshape(1, d))


def _tc_layer(h, agg, dinv, w, b, *, last, bn=512):
    n, d = h.shape
    grid = (pl.cdiv(n, bn),)
    return pl.pallas_call(
        functools.partial(_tc_layer_body, d=d, last=last),
        grid=grid,
        in_specs=[
            pl.BlockSpec((bn, d), lambda i: (i, 0)),
            pl.BlockSpec((_NC, bn, d), lambda i: (0, i, 0)),
            pl.BlockSpec((bn, d), lambda i: (i, 0)),
            pl.BlockSpec((2 * d, d), lambda i: (0, 0)),
            pl.BlockSpec((1, d), lambda i: (0, 0)),
        ],
        out_specs=pl.BlockSpec((bn, d), lambda i: (i, 0)),
        out_shape=jax.ShapeDtypeStruct((n, d), jnp.float32),
    )(h, agg, dinv, w, b.reshape(1, d))


def kernel(x, edge_index, W1, b1, W2, b2, W3, b3):
    n, d = x.shape
    e = edge_index.shape[1]
    nw = _NC * _NS
    n_chunks = e // (nw * _CHUNK)
    src = edge_index[0].reshape(nw, n_chunks, _CHUNK)
    dst = edge_index[1].reshape(nw, n_chunks, _CHUNK)

    degarr = _sc_degree(n, e, d)(dst)
    agg1 = _sc_segment_sum(n, e, d)(x, src, dst)
    h1, dinv = _tc_layer1(x, agg1, degarr, W1, b1)
    agg2 = _sc_segment_sum(n, e, d)(h1, src, dst)
    h2 = _tc_layer(h1, agg2, dinv, W2, b2, last=False)
    agg3 = _sc_segment_sum(n, e, d)(h2, src, dst)
    return _tc_layer(h2, agg3, dinv, W3, b3, last=True)
